# baseline (speedup 1.0000x reference)
import functools

import jax
import jax.numpy as jnp
from jax import lax
from jax.experimental import pallas as pl
from jax.experimental.pallas import tpu as pltpu
from jax.experimental.pallas import tpu_sc as plsc

_DIM = 64
_SCALE = 8.0
_COLS_BLOCK = 4096


def _transform_body(tt_ref, out_ref):
    pos = lax.broadcasted_iota(jnp.int32, (_COLS_BLOCK, _DIM), 1).astype(jnp.float32) + 1.0
    out_ref[:, : _DIM] = tt_ref[...].T * _SCALE + pos


def _transform(table_t):
    vocab = table_t.shape[1]
    return pl.pallas_call(
        _transform_body,
        grid=((vocab + _COLS_BLOCK - 1) // _COLS_BLOCK,),
        in_specs=[pl.BlockSpec((_DIM, _COLS_BLOCK), lambda i: (0, i))],
        out_specs=pl.BlockSpec((_COLS_BLOCK, 2 * _DIM), lambda i: (i, 0)),
        out_shape=jax.ShapeDtypeStruct((vocab, 2 * _DIM), jnp.float32),
    )(table_t)


@functools.lru_cache(maxsize=None)
def _make_gather(batch, seq, vocab):
    info = plsc.get_sparse_core_info()
    nc, ns = info.num_cores, info.num_subcores
    nw = nc * ns
    lanes = info.num_lanes
    bpw = batch // nw
    ndt = _DIM // 8
    mesh = plsc.VectorSubcoreMesh(core_axis_name="c", subcore_axis_name="s")

    @functools.partial(
        pl.kernel,
        mesh=mesh,
        compiler_params=pltpu.CompilerParams(
            use_tc_tiling_on_sc=False, needs_layout_passes=False
        ),
        out_type=jax.ShapeDtypeStruct((seq, ndt, batch // bpw, 8 * bpw), jnp.float32),
        scratch_types=[
            pltpu.VMEM((seq, bpw), jnp.int32),
            pltpu.VMEM((bpw, _DIM), jnp.float32),
            pltpu.VMEM((bpw, _DIM), jnp.float32),
            pltpu.VMEM((_DIM * bpw,), jnp.float32),
            pltpu.VMEM((_DIM * bpw,), jnp.float32),
            pltpu.SemaphoreType.DMA,
            pltpu.SemaphoreType.DMA,
            pltpu.SemaphoreType.DMA,
            pltpu.SemaphoreType.DMA,
        ],
    )
    def k(idx_hbm, table_hbm, out_hbm, idx_v, buf_a, buf_b, tbuf_a, tbuf_b,
          gs_a, gs_b, ws_a, ws_b):
        wid = lax.axis_index("s") * nc + lax.axis_index("c")
        pltpu.sync_copy(idx_hbm.at[wid], idx_v)
        bvecs = [lax.iota(jnp.int32, lanes) + g * lanes for g in range(bpw // lanes)]

        def start_gather(l, buf, sem):
            pltpu.async_copy(table_hbm.at[idx_v.at[l]], buf, sem)

        def wait_gather(l, buf, sem):
            pltpu.make_async_copy(table_hbm.at[idx_v.at[l]], buf, sem).wait()

        def transpose(buf, tbuf):
            def drow(d, carry):
                dsplat = jnp.full((lanes,), d, jnp.int32)
                for g in range(bpw // lanes):
                    v = plsc.load_gather(buf, [bvecs[g], dsplat])
                    tbuf[pl.ds(d * bpw + g * lanes, lanes)] = v
                return carry

            lax.fori_loop(0, _DIM, drow, 0)

        def start_writes(l, tbuf, sem):
            for dt in range(ndt):
                pltpu.async_copy(
                    tbuf.at[pl.ds(dt * 8 * bpw, 8 * bpw)], out_hbm.at[l, dt, wid], sem)

        def wait_writes(l, tbuf, sem):
            for dt in range(ndt):
                pltpu.make_async_copy(
                    tbuf.at[pl.ds(dt * 8 * bpw, 8 * bpw)], out_hbm.at[l, dt, wid], sem
                ).wait()

        start_gather(0, buf_a, gs_a)
        start_gather(1, buf_b, gs_b)

        def body(l2, carry):
            l = 2 * l2
            wait_gather(l, buf_a, gs_a)

            @pl.when(l >= 2)
            def _():
                wait_writes(l - 2, tbuf_a, ws_a)

            transpose(buf_a, tbuf_a)
            start_writes(l, tbuf_a, ws_a)

            @pl.when(l + 2 < seq)
            def _():
                start_gather(l + 2, buf_a, gs_a)

            wait_gather(l + 1, buf_b, gs_b)

            @pl.when(l >= 2)
            def _():
                wait_writes(l - 1, tbuf_b, ws_b)

            transpose(buf_b, tbuf_b)
            start_writes(l + 1, tbuf_b, ws_b)

            @pl.when(l + 3 < seq)
            def _():
                start_gather(l + 3, buf_b, gs_b)

            return carry

        lax.fori_loop(0, seq // 2, body, 0)
        wait_writes(seq - 2, tbuf_a, ws_a)
        wait_writes(seq - 1, tbuf_b, ws_b)

    return k


def kernel(x, table):
    b, l = x.shape
    nw = plsc.get_sparse_core_info().num_cores * plsc.get_sparse_core_info().num_subcores
    idx = (x.astype(jnp.int32) * 2).reshape(nw, b // nw, l).transpose(0, 2, 1)
    table2 = _transform(table.T).reshape(2 * table.shape[0], _DIM)
    out5 = _make_gather(b, l, 2 * table.shape[0])(idx, table2)
    return out5.reshape(1, l, _DIM, b).transpose(0, 3, 1, 2)


# speedup vs baseline: 1.7699x; 1.7699x over previous
import functools

import jax
import jax.numpy as jnp
from jax import lax
from jax.experimental import pallas as pl
from jax.experimental.pallas import tpu as pltpu
from jax.experimental.pallas import tpu_sc as plsc

_DIM = 64
_SCALE = 8.0
_COLS_BLOCK = 4096


def _transform_body(tt_ref, out_ref):
    pos = lax.broadcasted_iota(jnp.int32, (_COLS_BLOCK, _DIM), 1).astype(jnp.float32) + 1.0
    out_ref[:, : _DIM] = tt_ref[...].T * _SCALE + pos


def _transform(table_t):
    vocab = table_t.shape[1]
    return pl.pallas_call(
        _transform_body,
        grid=((vocab + _COLS_BLOCK - 1) // _COLS_BLOCK,),
        in_specs=[pl.BlockSpec((_DIM, _COLS_BLOCK), lambda i: (0, i))],
        out_specs=pl.BlockSpec((_COLS_BLOCK, 2 * _DIM), lambda i: (i, 0)),
        out_shape=jax.ShapeDtypeStruct((vocab, 2 * _DIM), jnp.float32),
    )(table_t)


@functools.lru_cache(maxsize=None)
def _make_gather(batch, seq, vocab):
    info = plsc.get_sparse_core_info()
    nc, ns = info.num_cores, info.num_subcores
    nw = nc * ns
    rows = batch * seq
    rpw = rows // nw
    chunk = 128
    nchunks = rpw // chunk
    mesh = plsc.VectorSubcoreMesh(core_axis_name="c", subcore_axis_name="s")

    @functools.partial(
        pl.kernel,
        mesh=mesh,
        compiler_params=pltpu.CompilerParams(use_tc_tiling_on_sc=False),
        out_type=jax.ShapeDtypeStruct((rows, _DIM), jnp.float32),
        scratch_types=[
            pltpu.VMEM((nchunks, chunk), jnp.int32),
            pltpu.VMEM((chunk, _DIM), jnp.float32),
            pltpu.VMEM((chunk, _DIM), jnp.float32),
            pltpu.SemaphoreType.DMA,
            pltpu.SemaphoreType.DMA,
            pltpu.SemaphoreType.DMA,
            pltpu.SemaphoreType.DMA,
        ],
    )
    def k(idx_hbm, table_hbm, out_hbm, idx_v, buf_a, buf_b, gs_a, gs_b, ws_a, ws_b):
        wid = lax.axis_index("s") * nc + lax.axis_index("c")
        r0 = wid * rpw
        pltpu.sync_copy(idx_hbm.at[wid], idx_v)

        def start_gather(j, buf, sem):
            pltpu.async_copy(table_hbm.at[idx_v.at[j]], buf, sem)

        def start_write(j, buf, sem):
            pltpu.async_copy(buf, out_hbm.at[pl.ds(r0 + j * chunk, chunk)], sem)

        start_gather(0, buf_a, gs_a)
        start_gather(1, buf_b, gs_b)

        def body(j2, carry):
            j = 2 * j2
            pltpu.make_async_copy(table_hbm.at[idx_v.at[j]], buf_a, gs_a).wait()
            start_write(j, buf_a, ws_a)
            pltpu.make_async_copy(table_hbm.at[idx_v.at[j + 1]], buf_b, gs_b).wait()
            start_write(j + 1, buf_b, ws_b)

            @pl.when(j + 2 < nchunks)
            def _():
                pltpu.make_async_copy(
                    buf_a, out_hbm.at[pl.ds(r0 + j * chunk, chunk)], ws_a).wait()
                start_gather(j + 2, buf_a, gs_a)

            @pl.when(j + 3 < nchunks)
            def _():
                pltpu.make_async_copy(
                    buf_b, out_hbm.at[pl.ds(r0 + (j + 1) * chunk, chunk)], ws_b).wait()
                start_gather(j + 3, buf_b, gs_b)

            return carry

        lax.fori_loop(0, nchunks // 2, body, 0)
        pltpu.make_async_copy(
            buf_a, out_hbm.at[pl.ds(r0 + (nchunks - 2) * chunk, chunk)], ws_a).wait()
        pltpu.make_async_copy(
            buf_b, out_hbm.at[pl.ds(r0 + (nchunks - 1) * chunk, chunk)], ws_b).wait()

    return k


def kernel(x, table):
    b, l = x.shape
    nw = plsc.get_sparse_core_info().num_cores * plsc.get_sparse_core_info().num_subcores
    idx = (x.astype(jnp.int32) * 2).reshape(nw, -1, 128)
    table2 = _transform(table.T).reshape(2 * table.shape[0], _DIM)
    out = _make_gather(b, l, 2 * table.shape[0])(idx, table2)
    return out.reshape(1, b, l, _DIM)


# speedup vs baseline: 2.6734x; 1.5105x over previous
import functools

import jax
import jax.numpy as jnp
from jax import lax
from jax.experimental import pallas as pl
from jax.experimental.pallas import tpu as pltpu
from jax.experimental.pallas import tpu_sc as plsc

_DIM = 64
_SCALE = 8.0
_COLS_BLOCK = 4096


def _transform_body(tt_ref, out_ref):
    pos = lax.broadcasted_iota(jnp.int32, (_COLS_BLOCK, _DIM), 1).astype(jnp.float32) + 1.0
    out_ref[:, : _DIM] = tt_ref[...].T * _SCALE + pos


def _transform(table_t):
    vocab = table_t.shape[1]
    return pl.pallas_call(
        _transform_body,
        grid=((vocab + _COLS_BLOCK - 1) // _COLS_BLOCK,),
        in_specs=[pl.BlockSpec((_DIM, _COLS_BLOCK), lambda i: (0, i))],
        out_specs=pl.BlockSpec((_COLS_BLOCK, 2 * _DIM), lambda i: (i, 0)),
        out_shape=jax.ShapeDtypeStruct((vocab, 2 * _DIM), jnp.float32),
    )(table_t)


_TB = 256


def _format_body(in_ref, out_ref):
    seq2 = in_ref.shape[0] // _TB
    inr = in_ref[...].reshape(_TB, seq2, 2 * _DIM)
    for l2 in range(seq2):
        st = inr[:, l2, :].T
        out_ref[0, 2 * l2] = st[:_DIM]
        out_ref[0, 2 * l2 + 1] = st[_DIM:]


def _format(lin2, batch, seq):
    return pl.pallas_call(
        _format_body,
        grid=(batch // _TB,),
        in_specs=[pl.BlockSpec((_TB * seq // 2, 2 * _DIM), lambda i: (i, 0))],
        out_specs=pl.BlockSpec((1, seq, _DIM, _TB), lambda i: (0, 0, 0, i)),
        out_shape=jax.ShapeDtypeStruct((1, seq, _DIM, batch), jnp.float32),
    )(lin2)


@functools.lru_cache(maxsize=None)
def _make_gather(batch, seq, vocab):
    info = plsc.get_sparse_core_info()
    nc, ns = info.num_cores, info.num_subcores
    nw = nc * ns
    rows = batch * seq
    rpw = rows // nw
    chunk = 128
    nchunks = rpw // chunk
    mesh = plsc.VectorSubcoreMesh(core_axis_name="c", subcore_axis_name="s")

    @functools.partial(
        pl.kernel,
        mesh=mesh,
        compiler_params=pltpu.CompilerParams(use_tc_tiling_on_sc=False),
        out_type=jax.ShapeDtypeStruct((rows, _DIM), jnp.float32),
        scratch_types=[
            pltpu.VMEM((nchunks, chunk), jnp.int32),
            pltpu.VMEM((chunk, _DIM), jnp.float32),
            pltpu.VMEM((chunk, _DIM), jnp.float32),
            pltpu.SemaphoreType.DMA,
            pltpu.SemaphoreType.DMA,
            pltpu.SemaphoreType.DMA,
            pltpu.SemaphoreType.DMA,
        ],
    )
    def k(idx_hbm, table_hbm, out_hbm, idx_v, buf_a, buf_b, gs_a, gs_b, ws_a, ws_b):
        wid = lax.axis_index("s") * nc + lax.axis_index("c")
        r0 = wid * rpw
        pltpu.sync_copy(idx_hbm.at[wid], idx_v)

        def start_gather(j, buf, sem):
            pltpu.async_copy(table_hbm.at[idx_v.at[j]], buf, sem)

        def start_write(j, buf, sem):
            pltpu.async_copy(buf, out_hbm.at[pl.ds(r0 + j * chunk, chunk)], sem)

        start_gather(0, buf_a, gs_a)
        start_gather(1, buf_b, gs_b)

        def body(j2, carry):
            j = 2 * j2
            pltpu.make_async_copy(table_hbm.at[idx_v.at[j]], buf_a, gs_a).wait()
            start_write(j, buf_a, ws_a)
            pltpu.make_async_copy(table_hbm.at[idx_v.at[j + 1]], buf_b, gs_b).wait()
            start_write(j + 1, buf_b, ws_b)

            @pl.when(j + 2 < nchunks)
            def _():
                pltpu.make_async_copy(
                    buf_a, out_hbm.at[pl.ds(r0 + j * chunk, chunk)], ws_a).wait()
                start_gather(j + 2, buf_a, gs_a)

            @pl.when(j + 3 < nchunks)
            def _():
                pltpu.make_async_copy(
                    buf_b, out_hbm.at[pl.ds(r0 + (j + 1) * chunk, chunk)], ws_b).wait()
                start_gather(j + 3, buf_b, gs_b)

            return carry

        lax.fori_loop(0, nchunks // 2, body, 0)
        pltpu.make_async_copy(
            buf_a, out_hbm.at[pl.ds(r0 + (nchunks - 2) * chunk, chunk)], ws_a).wait()
        pltpu.make_async_copy(
            buf_b, out_hbm.at[pl.ds(r0 + (nchunks - 1) * chunk, chunk)], ws_b).wait()

    return k


def kernel(x, table):
    b, l = x.shape
    nw = plsc.get_sparse_core_info().num_cores * plsc.get_sparse_core_info().num_subcores
    idx = (x.astype(jnp.int32) * 2).reshape(nw, -1, 128)
    table2 = _transform(table.T).reshape(2 * table.shape[0], _DIM)
    out = _make_gather(b, l, 2 * table.shape[0])(idx, table2)
    t4 = _format(out.reshape(b * l // 2, 2 * _DIM), b, l)
    return t4.transpose(0, 3, 1, 2)
